# Initial kernel scaffold; baseline (speedup 1.0000x reference)
import jax
import jax.numpy as jnp
from jax.experimental import pallas as pl


def kernel(x_path, x_omic1, x_omic2, x_omic3, x_omic4, x_omic5, x_omic6, params):
    raise NotImplementedError("write your pallas kernel here")


# speedup vs baseline: 10.0936x; 10.0936x over previous
import functools

import jax
import jax.numpy as jnp
import numpy as np
from jax import lax
from jax.experimental import pallas as pl
from jax.experimental.pallas import tpu as pltpu
from jax.experimental.pallas import tpu_sc as plsc

_DIM_IN = 1024
_DIM_H = 256
_N = 4096
_K = 6
_OT_REG = 0.1
_OT_TAU = 0.5
_FI = _OT_TAU / (_OT_TAU + _OT_REG)
_SCALE = _DIM_H ** (-0.5)
_RB = 256
_NRB = _N // _RB
_NEG = float(np.float32(-3.0e38))
_IMAX = np.int32(2**31 - 1)
_F32 = jnp.float32


def _lrelu(x):
    return jnp.where(x >= 0, x, 0.01 * x)


def _elu(x):
    return jnp.where(x > 0, x, jnp.exp(x) - 1.0)


def _sigmoid(x):
    return 1.0 / (1.0 + jnp.exp(-x))


def _dot(a, b):
    return jnp.dot(a, b, preferred_element_type=_F32)


def _dot_bt(a, b):
    return lax.dot_general(a, b, (((1,), (1,)), ((), ())),
                           preferred_element_type=_F32)


def _xp_body(x_ref, w_ref, b_ref, xp_ref, psum_ref):
    xp = _lrelu(_dot(x_ref[...], w_ref[...]) + b_ref[...])
    xp_ref[...] = xp
    psum_ref[...] = jnp.sum(xp, axis=0, keepdims=True)[None]


def _run_xp(x_path, fc1T, fc1b):
    return pl.pallas_call(
        _xp_body,
        grid=(_NRB,),
        in_specs=[
            pl.BlockSpec((_RB, _DIM_IN), lambda i: (i, 0)),
            pl.BlockSpec((_DIM_IN, _DIM_H), lambda i: (0, 0)),
            pl.BlockSpec((1, _DIM_H), lambda i: (0, 0)),
        ],
        out_specs=[
            pl.BlockSpec((_RB, _DIM_H), lambda i: (i, 0)),
            pl.BlockSpec((1, 1, _DIM_H), lambda i: (i, 0, 0)),
        ],
        out_shape=[
            jax.ShapeDtypeStruct((_N, _DIM_H), _F32),
            jax.ShapeDtypeStruct((_NRB, 1, _DIM_H), _F32),
        ],
    )(x_path, fc1T, fc1b)


def _et_body(xp_ref, psum_ref, wh_ref, bh_ref, wt_ref, bt_ref, eh_ref, et_ref):
    m = jnp.sum(psum_ref[...].reshape(_NRB, _DIM_H), axis=0,
                keepdims=True) * (1.0 / _N)
    xpm = (xp_ref[...] + m) * 0.5
    eh_ref[...] = _dot(xpm, wh_ref[...]) + bh_ref[...]
    et_ref[...] = _dot(xpm, wt_ref[...]) + bt_ref[...]


def _run_et(xp, psum, whT, bh, wtT, bt):
    return pl.pallas_call(
        _et_body,
        grid=(_NRB,),
        in_specs=[
            pl.BlockSpec((_RB, _DIM_H), lambda i: (i, 0)),
            pl.BlockSpec((_NRB, 1, _DIM_H), lambda i: (0, 0, 0)),
            pl.BlockSpec((_DIM_H, _DIM_H), lambda i: (0, 0)),
            pl.BlockSpec((1, _DIM_H), lambda i: (0, 0)),
            pl.BlockSpec((_DIM_H, _DIM_H), lambda i: (0, 0)),
            pl.BlockSpec((1, _DIM_H), lambda i: (0, 0)),
        ],
        out_specs=[
            pl.BlockSpec((_RB, _DIM_H), lambda i: (i, 0)),
            pl.BlockSpec((_RB, _DIM_H), lambda i: (i, 0)),
        ],
        out_shape=[
            jax.ShapeDtypeStruct((_N, _DIM_H), _F32),
            jax.ShapeDtypeStruct((_N, _DIM_H), _F32),
        ],
    )(xp, psum, whT, bh, wtT, bt)


def _topk_body(eh_ref, et_ref, w_ref, i_ref):
    vals = _dot_bt(eh_ref[...] * _SCALE, et_ref[...])
    iota = lax.broadcasted_iota(jnp.int32, (_RB, _N), 1)
    ws, idxs = [], []
    for _ in range(_K):
        m = jnp.max(vals, axis=1, keepdims=True)
        idx = jnp.min(jnp.where(vals == m, iota, _IMAX), axis=1, keepdims=True)
        ws.append(m)
        idxs.append(idx)
        vals = jnp.where(iota == idx, _NEG, vals)
    w_ref[...] = jnp.concatenate(ws, axis=1)
    i_ref[...] = jnp.concatenate(idxs, axis=1)


def _run_topk(eh, et):
    return pl.pallas_call(
        _topk_body,
        grid=(_NRB,),
        in_specs=[
            pl.BlockSpec((_RB, _DIM_H), lambda i: (i, 0)),
            pl.BlockSpec((_N, _DIM_H), lambda i: (0, 0)),
        ],
        out_specs=[
            pl.BlockSpec((_RB, _K), lambda i: (i, 0)),
            pl.BlockSpec((_RB, _K), lambda i: (i, 0)),
        ],
        out_shape=[
            jax.ShapeDtypeStruct((_N, _K), _F32),
            jax.ShapeDtypeStruct((_N, _K), jnp.int32),
        ],
    )(eh, et)


def _gather_rows(et, idx3d):
    info = plsc.get_sparse_core_info()
    nc = info.num_cores
    nw = nc * info.num_subcores
    nch = idx3d.shape[1]
    rows_per_w = nch * 128
    rows = nw * rows_per_w
    mesh = plsc.VectorSubcoreMesh(core_axis_name="c", subcore_axis_name="s")

    @functools.partial(
        pl.kernel,
        out_type=jax.ShapeDtypeStruct((rows, _DIM_H), _F32),
        mesh=mesh,
        scratch_types=[
            pltpu.VMEM((nch, 128), jnp.int32),
            pltpu.VMEM((128, _DIM_H), _F32),
            pltpu.SemaphoreType.DMA,
        ],
    )
    def gk(table_hbm, idx_hbm, out_hbm, idx_v, rows_v, sem):
        wid = lax.axis_index("s") * nc + lax.axis_index("c")
        pltpu.sync_copy(idx_hbm.at[wid], idx_v)
        for c in range(nch):
            pltpu.async_copy(table_hbm.at[idx_v.at[c]], rows_v, sem).wait()
            pltpu.sync_copy(
                rows_v, out_hbm.at[pl.ds(wid * rows_per_w + c * 128, 128)])

    return gk(et, idx3d)


def _comb_body(eh_ref, tw_ref, nb_ref, w1_ref, b1_ref, w2_ref, b2_ref,
               out_ref):
    eh = eh_ref[...]
    p = jax.nn.softmax(tw_ref[...], axis=1)
    nbs = []
    kas = []
    for k in range(_K):
        nbk = nb_ref[:, k, :]
        pk = p[:, k:k + 1]
        gk = jnp.tanh((2.0 - pk) * eh + pk * nbk)
        kas.append(jnp.sum(nbk, axis=1, keepdims=True) *
                   jnp.sum(gk, axis=1, keepdims=True))
        nbs.append(nbk)
    kp = jax.nn.softmax(jnp.concatenate(kas, axis=1), axis=1)
    enh = kp[:, 0:1] * nbs[0]
    for k in range(1, _K):
        enh = enh + kp[:, k:k + 1] * nbs[k]
    s = _lrelu(_dot(eh + enh, w1_ref[...]) + b1_ref[...])
    b = _lrelu(_dot(eh * enh, w2_ref[...]) + b2_ref[...])
    out_ref[...] = s + b


def _run_comb(eh, tw, nb3, lin1T, b1, lin2T, b2):
    return pl.pallas_call(
        _comb_body,
        grid=(_NRB,),
        in_specs=[
            pl.BlockSpec((_RB, _DIM_H), lambda i: (i, 0)),
            pl.BlockSpec((_RB, _K), lambda i: (i, 0)),
            pl.BlockSpec((_RB, _K, _DIM_H), lambda i: (i, 0, 0)),
            pl.BlockSpec((_DIM_H, _DIM_H), lambda i: (0, 0)),
            pl.BlockSpec((1, _DIM_H), lambda i: (0, 0)),
            pl.BlockSpec((_DIM_H, _DIM_H), lambda i: (0, 0)),
            pl.BlockSpec((1, _DIM_H), lambda i: (0, 0)),
        ],
        out_specs=pl.BlockSpec((_RB, _DIM_H), lambda i: (i, 0)),
        out_shape=jax.ShapeDtypeStruct((_N, _DIM_H), _F32),
    )(eh, tw, nb3, lin1T, b1, lin2T, b2)


def _omic_body(x_ref, w1_ref, b1_ref, w2_ref, b2_ref, out_ref):
    h = _elu(_dot(x_ref[0], w1_ref[0]) + b1_ref[0])
    out_ref[0] = _elu(_dot(h, w2_ref[0]) + b2_ref[0])


def _run_omic(xo3, w1T, b1, w2T, b2):
    smax = xo3.shape[2]
    out = pl.pallas_call(
        _omic_body,
        grid=(6,),
        in_specs=[
            pl.BlockSpec((1, 1, smax), lambda i: (i, 0, 0)),
            pl.BlockSpec((1, smax, _DIM_H), lambda i: (i, 0, 0)),
            pl.BlockSpec((1, 1, _DIM_H), lambda i: (i, 0, 0)),
            pl.BlockSpec((1, _DIM_H, _DIM_H), lambda i: (i, 0, 0)),
            pl.BlockSpec((1, 1, _DIM_H), lambda i: (i, 0, 0)),
        ],
        out_specs=pl.BlockSpec((1, 1, _DIM_H), lambda i: (i, 0, 0)),
        out_shape=jax.ShapeDtypeStruct((6, 1, _DIM_H), _F32),
    )(xo3, w1T, b1, w2T, b2)
    return out.reshape(6, _DIM_H)


def _ot_body(eh2_ref, ho_ref, ones_ref, out_ref):
    x = eh2_ref[...]
    x = x - jnp.min(x, axis=1, keepdims=True)
    y = ho_ref[...]
    y = y - jnp.min(y, axis=1, keepdims=True)
    ysq = jnp.sum(y * y, axis=1, keepdims=True)
    xsqT = _dot_bt(ones_ref[...], x * x)
    xy = _dot_bt(y, x)
    costT = (ysq - 2.0 * xy) + xsqT
    kt = jnp.exp(-(costT / jnp.max(costT)) * (1.0 / _OT_REG))
    a = 1.0 / _N
    b = 1.0 / 6.0

    def body(_, carry):
        u, v = carry
        kv = jnp.sum(kt * v, axis=0, keepdims=True)
        u = jnp.exp(_FI * jnp.log(a / (kv + 1e-16)))
        ku = jnp.sum(kt * u, axis=1, keepdims=True)
        v = jnp.exp(_FI * jnp.log(b / (ku + 1e-16)))
        return u, v

    u0 = jnp.full((1, _N), a, _F32)
    v0 = jnp.full((6, 1), b, _F32)
    u, v = lax.fori_loop(0, 100, body, (u0, v0))
    flowT = kt * u * v
    out_ref[...] = _dot(flowT, eh2_ref[...])


def _run_ot(eh2, ho):
    ones = jnp.ones((1, _DIM_H), _F32)
    return pl.pallas_call(
        _ot_body,
        out_shape=jax.ShapeDtypeStruct((6, _DIM_H), _F32),
    )(eh2, ho, ones)


def _head_body(hpc_ref, ho_ref, inw_ref, inb_ref, outw_ref, outb_ref,
               ln1g_ref, ln1b_ref, ff1w_ref, ff1b_ref, ff2w_ref, ff2b_ref,
               ln2g_ref, ln2b_ref, gaw_ref, gab_ref, gbw_ref, gbb_ref,
               gcw_ref, gcb_ref, rhow_ref, rhob_ref, mm1w_ref, mm1b_ref,
               mm2w_ref, mm2b_ref, clsw_ref, clsb_ref, out_ref):
    isq = float(1.0 / np.sqrt(32.0).astype(np.float32))

    def ln(x, g, b):
        m = jnp.mean(x, axis=-1, keepdims=True)
        v = jnp.mean((x - m) ** 2, axis=-1, keepdims=True)
        return (x - m) / jnp.sqrt(v + 1e-5) * g + b

    def enc(x, l):
        qkv = _dot(x, inw_ref[l]) + inb_ref[l]
        ohs = []
        for h in range(8):
            q = qkv[:, h * 32:(h + 1) * 32]
            k = qkv[:, 256 + h * 32:256 + (h + 1) * 32]
            v = qkv[:, 512 + h * 32:512 + (h + 1) * 32]
            att = jax.nn.softmax(_dot_bt(q, k) * isq, axis=-1)
            ohs.append(_dot(att, v))
        mo = _dot(jnp.concatenate(ohs, axis=1), outw_ref[l]) + outb_ref[l]
        x = ln(x + mo, ln1g_ref[l], ln1b_ref[l])
        ff = jnp.maximum(_dot(x, ff1w_ref[l]) + ff1b_ref[l], 0.0)
        ff = _dot(ff, ff2w_ref[l]) + ff2b_ref[l]
        return ln(x + ff, ln2g_ref[l], ln2b_ref[l])

    def pool(x, i):
        va = jnp.tanh(_dot(x, gaw_ref[i]) + gab_ref[i])
        vg = _sigmoid(_dot(x, gbw_ref[i]) + gbb_ref[i])
        s = jnp.sum(va * vg * gcw_ref[i], axis=1, keepdims=True) + gcb_ref[i]
        w = jax.nn.softmax(s, axis=0)
        pooled = jnp.sum(w * x, axis=0, keepdims=True)
        return jnp.maximum(_dot(pooled, rhow_ref[i]) + rhob_ref[i], 0.0)

    hp = enc(enc(hpc_ref[...], 0), 1)
    hom = enc(enc(ho_ref[...], 2), 3)
    h = jnp.concatenate([pool(hp, 0), pool(hom, 1)], axis=1)
    h = jnp.maximum(_dot(h, mm1w_ref[...]) + mm1b_ref[...], 0.0)
    h = jnp.maximum(_dot(h, mm2w_ref[...]) + mm2b_ref[...], 0.0)
    out_ref[...] = _dot(h, clsw_ref[...]) + clsb_ref[...]


def _run_head(hpc, ho, wts):
    return pl.pallas_call(
        _head_body,
        out_shape=jax.ShapeDtypeStruct((1, 4), _F32),
    )(hpc, ho, *wts)


def kernel(x_path, x_omic1, x_omic2, x_omic3, x_omic4, x_omic5, x_omic6,
           params):
    P = params
    omics = (x_omic1, x_omic2, x_omic3, x_omic4, x_omic5, x_omic6)
    smax = 600

    xp, psum = _run_xp(x_path, P['fc1_w'].T, P['fc1_b'][None])
    eh, et = _run_et(xp, psum, P['Wh_w'].T, P['Wh_b'][None],
                     P['Wt_w'].T, P['Wt_b'][None])
    tw, ti = _run_topk(eh, et)
    nb = _gather_rows(et, ti.reshape(32, 6, 128))
    eh2 = _run_comb(eh, tw, nb.reshape(_N, _K, _DIM_H),
                    P['lin1_w'].T, P['lin1_b'][None],
                    P['lin2_w'].T, P['lin2_b'][None])

    xo = jnp.zeros((6, 1, smax), _F32)
    w1T = jnp.zeros((6, smax, _DIM_H), _F32)
    for i in range(6):
        s = omics[i].shape[0]
        xo = xo.at[i, 0, :s].set(omics[i])
        w1T = w1T.at[i, :s, :].set(P['sig'][i]['w1'].T)
    b1 = jnp.stack([P['sig'][i]['b1'][None] for i in range(6)])
    w2T = jnp.stack([P['sig'][i]['w2'].T for i in range(6)])
    b2 = jnp.stack([P['sig'][i]['b2'][None] for i in range(6)])
    ho = _run_omic(xo, w1T, b1, w2T, b2)

    hpc = _run_ot(eh2, ho)

    encs = P['ptr'] + P['otr']
    wts = [
        jnp.stack([L['in_w'].T for L in encs]),
        jnp.stack([L['in_b'][None] for L in encs]),
        jnp.stack([L['out_w'].T for L in encs]),
        jnp.stack([L['out_b'][None] for L in encs]),
        jnp.stack([L['ln1_g'][None] for L in encs]),
        jnp.stack([L['ln1_b'][None] for L in encs]),
        jnp.stack([L['ff1_w'].T for L in encs]),
        jnp.stack([L['ff1_b'][None] for L in encs]),
        jnp.stack([L['ff2_w'].T for L in encs]),
        jnp.stack([L['ff2_b'][None] for L in encs]),
        jnp.stack([L['ln2_g'][None] for L in encs]),
        jnp.stack([L['ln2_b'][None] for L in encs]),
        jnp.stack([P['pa_a_w'].T, P['oa_a_w'].T]),
        jnp.stack([P['pa_a_b'][None], P['oa_a_b'][None]]),
        jnp.stack([P['pa_b_w'].T, P['oa_b_w'].T]),
        jnp.stack([P['pa_b_b'][None], P['oa_b_b'][None]]),
        jnp.stack([P['pa_c_w'], P['oa_c_w']]),
        jnp.stack([P['pa_c_b'][None], P['oa_c_b'][None]]),
        jnp.stack([P['prho_w'].T, P['orho_w'].T]),
        jnp.stack([P['prho_b'][None], P['orho_b'][None]]),
        P['mm1_w'].T, P['mm1_b'][None],
        P['mm2_w'].T, P['mm2_b'][None],
        P['cls_w'].T, P['cls_b'][None],
    ]
    return _run_head(hpc, ho, wts)


# speedup vs baseline: 10.8682x; 1.0767x over previous
import functools

import jax
import jax.numpy as jnp
import numpy as np
from jax import lax
from jax.experimental import pallas as pl
from jax.experimental.pallas import tpu as pltpu
from jax.experimental.pallas import tpu_sc as plsc

_DIM_IN = 1024
_DIM_H = 256
_N = 4096
_K = 6
_OT_REG = 0.1
_OT_TAU = 0.5
_FI = _OT_TAU / (_OT_TAU + _OT_REG)
_SCALE = _DIM_H ** (-0.5)
_RB = 256
_NRB = _N // _RB
_NEG = float(np.float32(-3.0e38))
_IMAX = np.int32(2**31 - 1)
_F32 = jnp.float32


def _lrelu(x):
    return jnp.where(x >= 0, x, 0.01 * x)


def _elu(x):
    return jnp.where(x > 0, x, jnp.exp(x) - 1.0)


def _sigmoid(x):
    return 1.0 / (1.0 + jnp.exp(-x))


def _dot(a, b):
    return jnp.dot(a, b, preferred_element_type=_F32)


def _dot_bt(a, b):
    return lax.dot_general(a, b, (((1,), (1,)), ((), ())),
                           preferred_element_type=_F32)


def _xp_body(x_ref, w_ref, b_ref, xp_ref, psum_ref):
    xp = _lrelu(_dot(x_ref[...], w_ref[...]) + b_ref[...])
    xp_ref[...] = xp
    psum_ref[...] = jnp.sum(xp, axis=0, keepdims=True)[None]


def _run_xp(x_path, fc1T, fc1b):
    return pl.pallas_call(
        _xp_body,
        grid=(_NRB,),
        in_specs=[
            pl.BlockSpec((_RB, _DIM_IN), lambda i: (i, 0)),
            pl.BlockSpec((_DIM_IN, _DIM_H), lambda i: (0, 0)),
            pl.BlockSpec((1, _DIM_H), lambda i: (0, 0)),
        ],
        out_specs=[
            pl.BlockSpec((_RB, _DIM_H), lambda i: (i, 0)),
            pl.BlockSpec((1, 1, _DIM_H), lambda i: (i, 0, 0)),
        ],
        out_shape=[
            jax.ShapeDtypeStruct((_N, _DIM_H), _F32),
            jax.ShapeDtypeStruct((_NRB, 1, _DIM_H), _F32),
        ],
    )(x_path, fc1T, fc1b)


def _et_body(xp_ref, psum_ref, wh_ref, bh_ref, wt_ref, bt_ref, eh_ref, et_ref):
    m = jnp.sum(psum_ref[...].reshape(_NRB, _DIM_H), axis=0,
                keepdims=True) * (1.0 / _N)
    xpm = (xp_ref[...] + m) * 0.5
    eh_ref[...] = _dot(xpm, wh_ref[...]) + bh_ref[...]
    et_ref[...] = _dot(xpm, wt_ref[...]) + bt_ref[...]


def _run_et(xp, psum, whT, bh, wtT, bt):
    return pl.pallas_call(
        _et_body,
        grid=(_NRB,),
        in_specs=[
            pl.BlockSpec((_RB, _DIM_H), lambda i: (i, 0)),
            pl.BlockSpec((_NRB, 1, _DIM_H), lambda i: (0, 0, 0)),
            pl.BlockSpec((_DIM_H, _DIM_H), lambda i: (0, 0)),
            pl.BlockSpec((1, _DIM_H), lambda i: (0, 0)),
            pl.BlockSpec((_DIM_H, _DIM_H), lambda i: (0, 0)),
            pl.BlockSpec((1, _DIM_H), lambda i: (0, 0)),
        ],
        out_specs=[
            pl.BlockSpec((_RB, _DIM_H), lambda i: (i, 0)),
            pl.BlockSpec((_RB, _DIM_H), lambda i: (i, 0)),
        ],
        out_shape=[
            jax.ShapeDtypeStruct((_N, _DIM_H), _F32),
            jax.ShapeDtypeStruct((_N, _DIM_H), _F32),
        ],
    )(xp, psum, whT, bh, wtT, bt)


def _topk_body(eh_ref, et_ref, w_ref, i_ref):
    vals = _dot_bt(eh_ref[...] * _SCALE, et_ref[...])
    bits = lax.bitcast_convert_type(vals, jnp.int32)
    skey = jnp.where(bits < 0, bits ^ jnp.int32(0x7FFFFFFF), bits)
    col = lax.broadcasted_iota(jnp.int32, (_RB, _N), 1)
    skey = (skey & jnp.int32(~0xFFF)) | (jnp.int32(_N - 1) - col)
    ws, idxs = [], []
    for _ in range(_K):
        m = jnp.max(skey, axis=1, keepdims=True)
        skey = jnp.where(skey == m, jnp.int32(-0x80000000), skey)
        idxs.append(jnp.int32(_N - 1) - (m & jnp.int32(0xFFF)))
        vb = m & jnp.int32(~0xFFF)
        vb = jnp.where(vb < 0, vb ^ jnp.int32(0x7FFFFFFF), vb)
        ws.append(lax.bitcast_convert_type(vb, jnp.float32))
    w_ref[...] = jnp.concatenate(ws, axis=1)
    i_ref[...] = jnp.concatenate(idxs, axis=1)


def _run_topk(eh, et):
    return pl.pallas_call(
        _topk_body,
        grid=(_NRB,),
        in_specs=[
            pl.BlockSpec((_RB, _DIM_H), lambda i: (i, 0)),
            pl.BlockSpec((_N, _DIM_H), lambda i: (0, 0)),
        ],
        out_specs=[
            pl.BlockSpec((_RB, _K), lambda i: (i, 0)),
            pl.BlockSpec((_RB, _K), lambda i: (i, 0)),
        ],
        out_shape=[
            jax.ShapeDtypeStruct((_N, _K), _F32),
            jax.ShapeDtypeStruct((_N, _K), jnp.int32),
        ],
    )(eh, et)


def _gather_rows(et, idx3d):
    info = plsc.get_sparse_core_info()
    nc = info.num_cores
    nw = nc * info.num_subcores
    nch = idx3d.shape[1]
    rows_per_w = nch * 128
    rows = nw * rows_per_w
    mesh = plsc.VectorSubcoreMesh(core_axis_name="c", subcore_axis_name="s")

    @functools.partial(
        pl.kernel,
        out_type=jax.ShapeDtypeStruct((rows, _DIM_H), _F32),
        mesh=mesh,
        scratch_types=[
            pltpu.VMEM((nch, 128), jnp.int32),
            pltpu.VMEM((3, 128, _DIM_H), _F32),
            pltpu.SemaphoreType.DMA,
            pltpu.SemaphoreType.DMA,
        ],
    )
    def gk(table_hbm, idx_hbm, out_hbm, idx_v, rows_v, gsem, wsem):
        wid = lax.axis_index("s") * nc + lax.axis_index("c")
        base = wid * rows_per_w
        pltpu.sync_copy(idx_hbm.at[wid], idx_v)
        nbuf = 3
        hg = [None] * nbuf
        hw = [None] * nbuf
        for c in range(min(nbuf - 1, nch)):
            hg[c] = pltpu.async_copy(
                table_hbm.at[idx_v.at[c]], rows_v.at[c], gsem)
        for c in range(nch):
            b = c % nbuf
            n = c + nbuf - 1
            if n < nch:
                bn = n % nbuf
                if hw[bn] is not None:
                    hw[bn].wait()
                hg[bn] = pltpu.async_copy(
                    table_hbm.at[idx_v.at[n]], rows_v.at[bn], gsem)
            hg[b].wait()
            hw[b] = pltpu.async_copy(
                rows_v.at[b], out_hbm.at[pl.ds(base + c * 128, 128)], wsem)
        for b in range(nbuf):
            if hw[b] is not None:
                hw[b].wait()

    return gk(et, idx3d)


def _comb_body(eh_ref, tw_ref, nb_ref, w1_ref, b1_ref, w2_ref, b2_ref,
               out_ref):
    eh = eh_ref[...]
    p = jax.nn.softmax(tw_ref[...], axis=1)
    nbs = []
    kas = []
    for k in range(_K):
        nbk = nb_ref[:, k, :]
        pk = p[:, k:k + 1]
        gk = jnp.tanh((2.0 - pk) * eh + pk * nbk)
        kas.append(jnp.sum(nbk, axis=1, keepdims=True) *
                   jnp.sum(gk, axis=1, keepdims=True))
        nbs.append(nbk)
    kp = jax.nn.softmax(jnp.concatenate(kas, axis=1), axis=1)
    enh = kp[:, 0:1] * nbs[0]
    for k in range(1, _K):
        enh = enh + kp[:, k:k + 1] * nbs[k]
    s = _lrelu(_dot(eh + enh, w1_ref[...]) + b1_ref[...])
    b = _lrelu(_dot(eh * enh, w2_ref[...]) + b2_ref[...])
    out_ref[...] = s + b


def _run_comb(eh, tw, nb3, lin1T, b1, lin2T, b2):
    return pl.pallas_call(
        _comb_body,
        grid=(_NRB,),
        in_specs=[
            pl.BlockSpec((_RB, _DIM_H), lambda i: (i, 0)),
            pl.BlockSpec((_RB, _K), lambda i: (i, 0)),
            pl.BlockSpec((_RB, _K, _DIM_H), lambda i: (i, 0, 0)),
            pl.BlockSpec((_DIM_H, _DIM_H), lambda i: (0, 0)),
            pl.BlockSpec((1, _DIM_H), lambda i: (0, 0)),
            pl.BlockSpec((_DIM_H, _DIM_H), lambda i: (0, 0)),
            pl.BlockSpec((1, _DIM_H), lambda i: (0, 0)),
        ],
        out_specs=pl.BlockSpec((_RB, _DIM_H), lambda i: (i, 0)),
        out_shape=jax.ShapeDtypeStruct((_N, _DIM_H), _F32),
    )(eh, tw, nb3, lin1T, b1, lin2T, b2)


def _omic_body(x_ref, w1_ref, b1_ref, w2_ref, b2_ref, out_ref):
    h = _elu(_dot(x_ref[0], w1_ref[0]) + b1_ref[0])
    out_ref[0] = _elu(_dot(h, w2_ref[0]) + b2_ref[0])


def _run_omic(xo3, w1T, b1, w2T, b2):
    smax = xo3.shape[2]
    out = pl.pallas_call(
        _omic_body,
        grid=(6,),
        in_specs=[
            pl.BlockSpec((1, 1, smax), lambda i: (i, 0, 0)),
            pl.BlockSpec((1, smax, _DIM_H), lambda i: (i, 0, 0)),
            pl.BlockSpec((1, 1, _DIM_H), lambda i: (i, 0, 0)),
            pl.BlockSpec((1, _DIM_H, _DIM_H), lambda i: (i, 0, 0)),
            pl.BlockSpec((1, 1, _DIM_H), lambda i: (i, 0, 0)),
        ],
        out_specs=pl.BlockSpec((1, 1, _DIM_H), lambda i: (i, 0, 0)),
        out_shape=jax.ShapeDtypeStruct((6, 1, _DIM_H), _F32),
    )(xo3, w1T, b1, w2T, b2)
    return out.reshape(6, _DIM_H)


def _ot_body(eh2_ref, ho_ref, ones_ref, out_ref):
    x = eh2_ref[...]
    x = x - jnp.min(x, axis=1, keepdims=True)
    y = ho_ref[...]
    y = y - jnp.min(y, axis=1, keepdims=True)
    ysq = jnp.sum(y * y, axis=1, keepdims=True)
    xsqT = _dot_bt(ones_ref[...], x * x)
    xy = _dot_bt(y, x)
    costT = (ysq - 2.0 * xy) + xsqT
    kt = jnp.exp(-(costT / jnp.max(costT)) * (1.0 / _OT_REG))
    a = 1.0 / _N
    b = 1.0 / 6.0

    def body(_, carry):
        u, v = carry
        kv = jnp.sum(kt * v, axis=0, keepdims=True)
        u = jnp.exp(_FI * jnp.log(a / (kv + 1e-16)))
        ku = jnp.sum(kt * u, axis=1, keepdims=True)
        v = jnp.exp(_FI * jnp.log(b / (ku + 1e-16)))
        return u, v

    u0 = jnp.full((1, _N), a, _F32)
    v0 = jnp.full((6, 1), b, _F32)
    u, v = lax.fori_loop(0, 100, body, (u0, v0))
    flowT = kt * u * v
    out_ref[...] = _dot(flowT, eh2_ref[...])


def _run_ot(eh2, ho):
    ones = jnp.ones((1, _DIM_H), _F32)
    return pl.pallas_call(
        _ot_body,
        out_shape=jax.ShapeDtypeStruct((6, _DIM_H), _F32),
    )(eh2, ho, ones)


def _head_body(hpc_ref, ho_ref, inw_ref, inb_ref, outw_ref, outb_ref,
               ln1g_ref, ln1b_ref, ff1w_ref, ff1b_ref, ff2w_ref, ff2b_ref,
               ln2g_ref, ln2b_ref, gaw_ref, gab_ref, gbw_ref, gbb_ref,
               gcw_ref, gcb_ref, rhow_ref, rhob_ref, mm1w_ref, mm1b_ref,
               mm2w_ref, mm2b_ref, clsw_ref, clsb_ref, out_ref):
    isq = float(1.0 / np.sqrt(32.0).astype(np.float32))

    def ln(x, g, b):
        m = jnp.mean(x, axis=-1, keepdims=True)
        v = jnp.mean((x - m) ** 2, axis=-1, keepdims=True)
        return (x - m) / jnp.sqrt(v + 1e-5) * g + b

    def enc(x, l):
        qkv = _dot(x, inw_ref[l]) + inb_ref[l]
        ohs = []
        for h in range(8):
            q = qkv[:, h * 32:(h + 1) * 32]
            k = qkv[:, 256 + h * 32:256 + (h + 1) * 32]
            v = qkv[:, 512 + h * 32:512 + (h + 1) * 32]
            att = jax.nn.softmax(_dot_bt(q, k) * isq, axis=-1)
            ohs.append(_dot(att, v))
        mo = _dot(jnp.concatenate(ohs, axis=1), outw_ref[l]) + outb_ref[l]
        x = ln(x + mo, ln1g_ref[l], ln1b_ref[l])
        ff = jnp.maximum(_dot(x, ff1w_ref[l]) + ff1b_ref[l], 0.0)
        ff = _dot(ff, ff2w_ref[l]) + ff2b_ref[l]
        return ln(x + ff, ln2g_ref[l], ln2b_ref[l])

    def pool(x, i):
        va = jnp.tanh(_dot(x, gaw_ref[i]) + gab_ref[i])
        vg = _sigmoid(_dot(x, gbw_ref[i]) + gbb_ref[i])
        s = jnp.sum(va * vg * gcw_ref[i], axis=1, keepdims=True) + gcb_ref[i]
        w = jax.nn.softmax(s, axis=0)
        pooled = jnp.sum(w * x, axis=0, keepdims=True)
        return jnp.maximum(_dot(pooled, rhow_ref[i]) + rhob_ref[i], 0.0)

    hp = enc(enc(hpc_ref[...], 0), 1)
    hom = enc(enc(ho_ref[...], 2), 3)
    h = jnp.concatenate([pool(hp, 0), pool(hom, 1)], axis=1)
    h = jnp.maximum(_dot(h, mm1w_ref[...]) + mm1b_ref[...], 0.0)
    h = jnp.maximum(_dot(h, mm2w_ref[...]) + mm2b_ref[...], 0.0)
    out_ref[...] = _dot(h, clsw_ref[...]) + clsb_ref[...]


def _run_head(hpc, ho, wts):
    return pl.pallas_call(
        _head_body,
        out_shape=jax.ShapeDtypeStruct((1, 4), _F32),
    )(hpc, ho, *wts)


def kernel(x_path, x_omic1, x_omic2, x_omic3, x_omic4, x_omic5, x_omic6,
           params):
    P = params
    omics = (x_omic1, x_omic2, x_omic3, x_omic4, x_omic5, x_omic6)
    smax = 600

    xp, psum = _run_xp(x_path, P['fc1_w'].T, P['fc1_b'][None])
    eh, et = _run_et(xp, psum, P['Wh_w'].T, P['Wh_b'][None],
                     P['Wt_w'].T, P['Wt_b'][None])
    tw, ti = _run_topk(eh, et)
    nb = _gather_rows(et, ti.reshape(32, 6, 128))
    eh2 = _run_comb(eh, tw, nb.reshape(_N, _K, _DIM_H),
                    P['lin1_w'].T, P['lin1_b'][None],
                    P['lin2_w'].T, P['lin2_b'][None])

    xo = jnp.zeros((6, 1, smax), _F32)
    w1T = jnp.zeros((6, smax, _DIM_H), _F32)
    for i in range(6):
        s = omics[i].shape[0]
        xo = xo.at[i, 0, :s].set(omics[i])
        w1T = w1T.at[i, :s, :].set(P['sig'][i]['w1'].T)
    b1 = jnp.stack([P['sig'][i]['b1'][None] for i in range(6)])
    w2T = jnp.stack([P['sig'][i]['w2'].T for i in range(6)])
    b2 = jnp.stack([P['sig'][i]['b2'][None] for i in range(6)])
    ho = _run_omic(xo, w1T, b1, w2T, b2)

    hpc = _run_ot(eh2, ho)

    encs = P['ptr'] + P['otr']
    wts = [
        jnp.stack([L['in_w'].T for L in encs]),
        jnp.stack([L['in_b'][None] for L in encs]),
        jnp.stack([L['out_w'].T for L in encs]),
        jnp.stack([L['out_b'][None] for L in encs]),
        jnp.stack([L['ln1_g'][None] for L in encs]),
        jnp.stack([L['ln1_b'][None] for L in encs]),
        jnp.stack([L['ff1_w'].T for L in encs]),
        jnp.stack([L['ff1_b'][None] for L in encs]),
        jnp.stack([L['ff2_w'].T for L in encs]),
        jnp.stack([L['ff2_b'][None] for L in encs]),
        jnp.stack([L['ln2_g'][None] for L in encs]),
        jnp.stack([L['ln2_b'][None] for L in encs]),
        jnp.stack([P['pa_a_w'].T, P['oa_a_w'].T]),
        jnp.stack([P['pa_a_b'][None], P['oa_a_b'][None]]),
        jnp.stack([P['pa_b_w'].T, P['oa_b_w'].T]),
        jnp.stack([P['pa_b_b'][None], P['oa_b_b'][None]]),
        jnp.stack([P['pa_c_w'], P['oa_c_w']]),
        jnp.stack([P['pa_c_b'][None], P['oa_c_b'][None]]),
        jnp.stack([P['prho_w'].T, P['orho_w'].T]),
        jnp.stack([P['prho_b'][None], P['orho_b'][None]]),
        P['mm1_w'].T, P['mm1_b'][None],
        P['mm2_w'].T, P['mm2_b'][None],
        P['cls_w'].T, P['cls_b'][None],
    ]
    return _run_head(hpc, ho, wts)
